# baseline (speedup 1.0000x reference)
import functools

import jax
import jax.numpy as jnp
from jax import lax
from jax.experimental import pallas as pl
from jax.experimental.pallas import tpu as pltpu, tpu_sc as plsc

B = 16384
F = 26
D = 32
NC = 2
NS = 16
NW = NC * NS
L = 16

ROWS = B * F
RPW = ROWS // NW
PAT = 208
GROUPS = RPW // PAT
C = 832
NCH = RPW // C


def _body(idx_hbm, pat_hbm, table_hbm, out_hbm,
          idx_v, pat_v, buf0, buf1, gsem, ssem):
    wid = lax.axis_index("s") * NC + lax.axis_index("c")
    base = wid * RPW

    pltpu.sync_copy(idx_hbm.at[pl.ds(base, RPW)], idx_v)
    pltpu.sync_copy(pat_hbm, pat_v)

    pat_regs = [pat_v[pl.ds(j * L, L)] for j in range(PAT // L)]

    def add_group(g, carry):
        s0 = g * PAT
        for j in range(PAT // L):
            sl = pl.ds(s0 + j * L, L)
            idx_v[sl] = idx_v[sl] + pat_regs[j]
        return carry

    lax.fori_loop(0, GROUPS, add_group, 0)

    bufs = [buf0, buf1]
    for k in range(NCH):
        buf = bufs[k % 2]
        pltpu.async_copy(
            table_hbm.at[idx_v.at[pl.ds(k * C, C)]], buf, gsem
        ).wait()
        pltpu.sync_copy(buf, out_hbm.at[pl.ds(base + k * C, C)])


@jax.jit
def _run(idx_flat, pat, table):
    mesh = plsc.VectorSubcoreMesh(core_axis_name="c", subcore_axis_name="s")
    return pl.kernel(
        _body,
        out_type=jax.ShapeDtypeStruct((ROWS, D), jnp.float32),
        mesh=mesh,
        scratch_types=[
            pltpu.VMEM((RPW,), jnp.int32),
            pltpu.VMEM((PAT,), jnp.int32),
            pltpu.VMEM((C, D), jnp.float32),
            pltpu.VMEM((C, D), jnp.float32),
            pltpu.SemaphoreType.DMA,
            pltpu.SemaphoreType.DMA,
        ],
        compiler_params=pltpu.CompilerParams(use_tc_tiling_on_sc=False),
    )(idx_flat, pat, table)


def kernel(sparse_indices, offsets, embed_table):
    idx_flat = sparse_indices.reshape(ROWS)
    pat = jnp.tile(offsets.reshape(F), L // 2)
    out = _run(idx_flat, pat, embed_table)
    return out.reshape(B, F, D)
